# baseline (speedup 1.0000x reference)
import functools

import jax
import jax.numpy as jnp
from jax import lax
from jax.experimental import pallas as pl
from jax.experimental.pallas import tpu as pltpu
from jax.experimental.pallas import tpu_sc as plsc

_D = 64
_SCALE = 8.0
_NC, _NS = 2, 16
_NW = _NC * _NS
_CHUNK = 128
_K = 4
_SUP = _CHUNK * _K


@functools.lru_cache(maxsize=None)
def _make_emb(nh, nb):
    assert nb == _SUP * _NW and nh % 2 == 0
    nit = nb // _CHUNK
    run = _K * 8 * _CHUNK
    mesh = plsc.VectorSubcoreMesh(core_axis_name="c", subcore_axis_name="s")

    @functools.partial(
        pl.kernel,
        out_type=jax.ShapeDtypeStruct((nh, _D // 8, nit, 8, _CHUNK), jnp.float32),
        mesh=mesh,
        compiler_params=pltpu.CompilerParams(
            use_tc_tiling_on_sc=False, needs_layout_passes=False
        ),
        scratch_types=[
            pltpu.VMEM((2, _K, _CHUNK), jnp.int32),
            pltpu.VMEM((2 * _SUP, _D), jnp.float32),
            pltpu.VMEM((_K, _D // 8, 1, 8, _CHUNK), jnp.float32),
            pltpu.SemaphoreType.DMA,
            pltpu.SemaphoreType.DMA,
        ],
    )
    def emb(idx_hbm, table_hbm, out_hbm, idx_v, rows_v, t_v, sem0, sem1):
        wid = lax.axis_index("s") * _NC + lax.axis_index("c")
        it0 = wid * _K

        def fire(h, b, sem):
            pltpu.sync_copy(idx_hbm.at[h, pl.ds(it0, _K)], idx_v.at[b])
            for j in range(_K):
                pltpu.async_copy(
                    table_hbm.at[idx_v.at[b, j]],
                    rows_v.at[pl.ds(b * _SUP + j * _CHUNK, _CHUNK)],
                    sem,
                )

        def drain(sem):
            pltpu.make_async_copy(
                table_hbm.at[pl.ds(0, _SUP)], rows_v.at[pl.ds(0, _SUP)], sem
            ).wait()

        def scale_store(h, boff):
            for jt in range(_K):

                @plsc.parallel_loop(0, _CHUNK, step=16, unroll=4)
                def _transpose(ic, jt=jt):
                    rows = lax.iota(jnp.int32, 16) + (boff + jt * _CHUNK + ic)
                    for k in range(_D):
                        v = plsc.load_gather(
                            rows_v, [rows, jnp.full((16,), k, jnp.int32)]
                        )
                        t_v[jt, k // 8, 0, k % 8, pl.ds(ic, 16)] = v * _SCALE

            for jt in range(_K):
                pltpu.sync_copy(
                    t_v.at[jt], out_hbm.at[h, :, pl.ds(it0 + jt, 1)]
                )

        fire(0, 0, sem0)

        def body(h, carry):
            par = lax.bitwise_and(h, 1)
            hn = jnp.minimum(h + 1, nh - 1)

            @pl.when(par == 0)
            def _():
                fire(hn, 1, sem1)
                drain(sem0)

            @pl.when(par == 1)
            def _():
                fire(hn, 0, sem0)
                drain(sem1)

            scale_store(h, par * _SUP)
            return carry

        lax.fori_loop(0, nh, body, 0)
        drain(sem0)

    return emb


def kernel(x, table):
    nb, nh = x.shape
    nit = nb // _CHUNK
    xt = x.T.reshape(nh, nit, _CHUNK).astype(jnp.int32)
    out5 = _make_emb(nh, nb)(xt, table)
    return jnp.transpose(out5, (2, 4, 0, 1, 3)).reshape(nb, nh, _D)


# speedup vs baseline: 3.6547x; 3.6547x over previous
import functools

import jax
import jax.numpy as jnp
from jax import lax
from jax.experimental import pallas as pl
from jax.experimental.pallas import tpu as pltpu
from jax.experimental.pallas import tpu_sc as plsc

_D = 64
_SCALE = 8.0
_NC, _NS = 2, 16
_NW = _NC * _NS
_CHUNK = 128
_K = 4
_SUP = _CHUNK * _K


@functools.lru_cache(maxsize=None)
def _make_emb(nh, nb):
    assert nb == _SUP * _NW and nh % 2 == 0
    nit = nb // _CHUNK
    run = _K * 8 * _CHUNK
    mesh = plsc.VectorSubcoreMesh(core_axis_name="c", subcore_axis_name="s")

    @functools.partial(
        pl.kernel,
        out_type=jax.ShapeDtypeStruct((nh, _D // 8, nit, 8, _CHUNK), jnp.float32),
        mesh=mesh,
        compiler_params=pltpu.CompilerParams(
            use_tc_tiling_on_sc=False, needs_layout_passes=False
        ),
        scratch_types=[
            pltpu.VMEM((2, _K, _CHUNK), jnp.int32),
            pltpu.VMEM((2 * _SUP, _D), jnp.float32),
            pltpu.VMEM((_D // 8, _K, 8, _CHUNK + 1), jnp.float32),
            pltpu.SemaphoreType.DMA,
            pltpu.SemaphoreType.DMA,
        ],
    )
    def emb(idx_hbm, table_hbm, out_hbm, idx_v, rows_v, t_v, sem0, sem1):
        wid = lax.axis_index("s") * _NC + lax.axis_index("c")
        it0 = wid * _K

        def fire(h, b, sem):
            pltpu.sync_copy(idx_hbm.at[h, pl.ds(it0, _K)], idx_v.at[b])
            for j in range(_K):
                pltpu.async_copy(
                    table_hbm.at[idx_v.at[b, j]],
                    rows_v.at[pl.ds(b * _SUP + j * _CHUNK, _CHUNK)],
                    sem,
                )

        def drain(sem):
            pltpu.make_async_copy(
                table_hbm.at[pl.ds(0, _SUP)],
                rows_v.at[pl.ds(0, _SUP)],
                sem,
            ).wait()

        def scale_store(h, boff):
            kvs = [lax.iota(jnp.int32, 16) + c * 16 for c in range(_D // 16)]
            ktvs = [kv >> 3 for kv in kvs]
            krvs = [kv & 7 for kv in kvs]

            @plsc.parallel_loop(0, _SUP, step=1, unroll=8)
            def _transpose(r):
                jtv = jnp.full((16,), r >> 7, jnp.int32)
                icv = jnp.full((16,), r & 127, jnp.int32)
                for c in range(_D // 16):
                    v = rows_v[boff + r, pl.ds(c * 16, 16)] * _SCALE
                    plsc.store_scatter(t_v, [ktvs[c], jtv, krvs[c], icv], v)

            for jt in range(_K):
                pltpu.sync_copy(
                    t_v.at[:, pl.ds(jt, 1), :, pl.ds(0, _CHUNK)],
                    out_hbm.at[h, :, pl.ds(it0 + jt, 1)],
                )

        fire(0, 0, sem0)

        def body(h, carry):
            par = lax.bitwise_and(h, 1)
            hn = jnp.minimum(h + 1, nh - 1)

            @pl.when(par == 0)
            def _():
                fire(hn, 1, sem1)
                drain(sem0)

            @pl.when(par == 1)
            def _():
                fire(hn, 0, sem0)
                drain(sem1)

            scale_store(h, par * _SUP)
            return carry

        lax.fori_loop(0, nh, body, 0)
        drain(sem0)

    return emb


def kernel(x, table):
    nb, nh = x.shape
    nit = nb // _CHUNK
    xt = x.T.reshape(nh, nit, _CHUNK).astype(jnp.int32)
    out5 = _make_emb(nh, nb)(xt, table)
    return jnp.transpose(out5, (2, 4, 0, 1, 3)).reshape(nb, nh, _D)


# speedup vs baseline: 3.7701x; 1.0316x over previous
import functools

import jax
import jax.numpy as jnp
from jax import lax
from jax.experimental import pallas as pl
from jax.experimental.pallas import tpu as pltpu
from jax.experimental.pallas import tpu_sc as plsc

_D = 64
_SCALE = 8.0
_NC, _NS = 2, 16
_NW = _NC * _NS
_CHUNK = 128
_K = 4
_SUP = _CHUNK * _K


@functools.lru_cache(maxsize=None)
def _make_emb(nh, nb):
    assert nb == _SUP * _NW and nh % 2 == 0
    nit = nb // _CHUNK
    run = _K * 8 * _CHUNK
    mesh = plsc.VectorSubcoreMesh(core_axis_name="c", subcore_axis_name="s")

    @functools.partial(
        pl.kernel,
        out_type=jax.ShapeDtypeStruct((nh, _D // 8, nit, 8, _CHUNK), jnp.float32),
        mesh=mesh,
        compiler_params=pltpu.CompilerParams(
            use_tc_tiling_on_sc=False, needs_layout_passes=False
        ),
        scratch_types=[
            pltpu.VMEM((2, _K, _CHUNK), jnp.int32),
            pltpu.VMEM((2 * _SUP, _D), jnp.float32),
            pltpu.VMEM((_D // 8, _K, 8, _CHUNK + 1), jnp.float32),
            pltpu.SemaphoreType.DMA,
            pltpu.SemaphoreType.DMA,
            pltpu.SemaphoreType.DMA,
        ],
    )
    def emb(idx_hbm, table_hbm, out_hbm, idx_v, rows_v, t_v, sem0, sem1, sem2):
        wid = lax.axis_index("s") * _NC + lax.axis_index("c")
        it0 = wid * _K

        def fire(h, b, sem):
            pltpu.sync_copy(idx_hbm.at[h, pl.ds(it0, _K)], idx_v.at[b])
            for j in range(_K):
                pltpu.async_copy(
                    table_hbm.at[idx_v.at[b, j]],
                    rows_v.at[pl.ds(b * _SUP + j * _CHUNK, _CHUNK)],
                    sem,
                )

        def drain(sem):
            pltpu.make_async_copy(
                table_hbm.at[pl.ds(0, _SUP)],
                rows_v.at[pl.ds(0, _SUP)],
                sem,
            ).wait()

        def drain_stores():
            for jt in range(_K):
                pltpu.make_async_copy(
                    out_hbm.at[0, :, pl.ds(0, 1)],
                    t_v.at[:, pl.ds(jt, 1), :, pl.ds(0, _CHUNK)],
                    sem2,
                ).wait()

        def scale_store(h, boff):
            kvs = [lax.iota(jnp.int32, 16) + c * 16 for c in range(_D // 16)]
            ktvs = [kv >> 3 for kv in kvs]
            krvs = [kv & 7 for kv in kvs]

            @plsc.parallel_loop(0, _SUP, step=1, unroll=8)
            def _transpose(r):
                jtv = jnp.full((16,), r >> 7, jnp.int32)
                icv = jnp.full((16,), r & 127, jnp.int32)
                for c in range(_D // 16):
                    v = rows_v[boff + r, pl.ds(c * 16, 16)] * _SCALE
                    plsc.store_scatter(t_v, [ktvs[c], jtv, krvs[c], icv], v)

            for jt in range(_K):
                pltpu.async_copy(
                    t_v.at[:, pl.ds(jt, 1), :, pl.ds(0, _CHUNK)],
                    out_hbm.at[h, :, pl.ds(it0 + jt, 1)],
                    sem2,
                )

        fire(0, 0, sem0)

        def body(h, carry):
            par = lax.bitwise_and(h, 1)
            hn = jnp.minimum(h + 1, nh - 1)

            @pl.when(par == 0)
            def _():
                fire(hn, 1, sem1)
                drain(sem0)

            @pl.when(par == 1)
            def _():
                fire(hn, 0, sem0)
                drain(sem1)

            @pl.when(h > 0)
            def _():
                drain_stores()

            scale_store(h, par * _SUP)
            return carry

        lax.fori_loop(0, nh, body, 0)
        drain(sem0)
        drain_stores()

    return emb


def kernel(x, table):
    nb, nh = x.shape
    nit = nb // _CHUNK
    xt = x.T.reshape(nh, nit, _CHUNK).astype(jnp.int32)
    out5 = _make_emb(nh, nb)(xt, table)
    return jnp.transpose(out5, (2, 4, 0, 1, 3)).reshape(nb, nh, _D)


# speedup vs baseline: 4.5188x; 1.1986x over previous
import functools

import jax
import jax.numpy as jnp
from jax import lax
from jax.experimental import pallas as pl
from jax.experimental.pallas import tpu as pltpu
from jax.experimental.pallas import tpu_sc as plsc

_D = 64
_SCALE = 8.0
_NC, _NS = 2, 16
_NW = _NC * _NS
_CHUNK = 128
_K = 4
_SUP = _CHUNK * _K
_TP = _K * _CHUNK + 3


@functools.lru_cache(maxsize=None)
def _make_emb(nh, nb):
    assert nb == _SUP * _NW and nh % 2 == 0
    nit = nb // _CHUNK
    mesh = plsc.VectorSubcoreMesh(core_axis_name="c", subcore_axis_name="s")

    @functools.partial(
        pl.kernel,
        out_type=jax.ShapeDtypeStruct((nh, _D // 8, nit, 8, _CHUNK), jnp.float32),
        mesh=mesh,
        compiler_params=pltpu.CompilerParams(
            use_tc_tiling_on_sc=False, needs_layout_passes=False
        ),
        scratch_types=[
            pltpu.VMEM((2, _K, _CHUNK), jnp.int32),
            pltpu.VMEM((2 * _SUP, _D), jnp.float32),
            pltpu.VMEM((_D // 8, 1, 8, _TP), jnp.float32),
            pltpu.SemaphoreType.DMA,
            pltpu.SemaphoreType.DMA,
            pltpu.SemaphoreType.DMA,
            pltpu.SemaphoreType.DMA,
            pltpu.SemaphoreType.DMA,
        ],
    )
    def emb(idx_hbm, table_hbm, out_hbm, idx_v, rows_v, t_v, g0, g1, so, i0, i1):
        gsem = (g0, g1)
        isem = (i0, i1)
        wid = lax.axis_index("s") * _NC + lax.axis_index("c")
        it0 = wid * _K

        def idx_fetch(h, b, sync=False):
            src, dst = idx_hbm.at[h, pl.ds(it0, _K)], idx_v.at[b]
            if sync:
                pltpu.sync_copy(src, dst)
            else:
                pltpu.async_copy(src, dst, isem[b])

        def idx_wait(b):
            pltpu.make_async_copy(
                idx_hbm.at[0, pl.ds(0, _K)], idx_v.at[b], isem[b]
            ).wait()

        def fire(b):
            for j in range(_K):
                pltpu.async_copy(
                    table_hbm.at[idx_v.at[b, j]],
                    rows_v.at[pl.ds(b * _SUP + j * _CHUNK, _CHUNK)],
                    gsem[b],
                )

        def drain(b):
            pltpu.make_async_copy(
                table_hbm.at[pl.ds(0, _SUP)],
                rows_v.at[pl.ds(b * _SUP, _SUP)],
                gsem[b],
            ).wait()

        def drain_stores():
            for jt in range(_K):
                pltpu.make_async_copy(
                    out_hbm.at[0, :, pl.ds(0, 1)],
                    t_v.at[:, :, :, pl.ds(0, _CHUNK)],
                    so,
                ).wait()

        def scale_store(h, boff):
            kvs = [lax.iota(jnp.int32, 16) + c * 16 for c in range(_D // 16)]
            ktvs = [kv >> 3 for kv in kvs]
            krvs = [kv & 7 for kv in kvs]
            zv = jnp.zeros((16,), jnp.int32)

            @plsc.parallel_loop(0, _SUP, step=1, unroll=16)
            def _transpose(r):
                rv = jnp.full((16,), r, jnp.int32)
                for c in range(_D // 16):
                    v = rows_v[boff + r, pl.ds(c * 16, 16)] * _SCALE
                    plsc.store_scatter(t_v, [ktvs[c], zv, krvs[c], rv], v)

            for jt in range(_K):
                pltpu.async_copy(
                    t_v.at[:, :, :, pl.ds(jt * _CHUNK, _CHUNK)],
                    out_hbm.at[h, :, pl.ds(it0 + jt, 1)],
                    so,
                )

        idx_fetch(0, 0, sync=True)
        idx_fetch(1, 1)
        fire(0)

        def body(h, carry):
            par = lax.bitwise_and(h, 1)
            hn2 = jnp.minimum(h + 2, nh - 1)

            @pl.when(par == 0)
            def _():
                idx_wait(1)
                fire(1)
                drain(0)
                idx_fetch(hn2, 0)

            @pl.when(par == 1)
            def _():
                idx_wait(0)
                fire(0)
                drain(1)
                idx_fetch(hn2, 1)

            @pl.when(h > 0)
            def _():
                drain_stores()

            scale_store(h, par * _SUP)
            return carry

        lax.fori_loop(0, nh, body, 0)
        drain(0)
        idx_wait(1)
        drain_stores()

    return emb


def kernel(x, table):
    nb, nh = x.shape
    nit = nb // _CHUNK
    xt = x.T.reshape(nh, nit, _CHUNK).astype(jnp.int32)
    out5 = _make_emb(nh, nb)(xt, table)
    return jnp.transpose(out5, (2, 4, 0, 1, 3)).reshape(nb, nh, _D)
